# baseline (speedup 1.0000x reference)
import functools

import jax
import jax.numpy as jnp
from jax import lax
from jax.experimental import pallas as pl
from jax.experimental.pallas import tpu as pltpu
from jax.experimental.pallas import tpu_sc as plsc

_NUM_SEGMENTS = 10000
_SP = 10240
_CHUNK = 80


def _h_body(emb_ref, conc_ref, roles_ref, w1e_ref, wcr_ref, b1_ref, h_ref):
    h = lax.dot_general(emb_ref[...].astype(jnp.bfloat16), w1e_ref[...],
                        (((1,), (0,)), ((), ())),
                        preferred_element_type=jnp.float32)
    h = h + conc_ref[...] * wcr_ref[0:1, :] + roles_ref[...] * wcr_ref[1:2, :]
    h_ref[...] = jnp.maximum(h + b1_ref[...], 0.0)


def _mlp_body(parts_ref, w2_ref, b2_ref, w3_ref, b3_ref, out_ref, *, s):
    pooled = parts_ref[0, 0:s, :] + parts_ref[1, 0:s, :]
    x = lax.dot_general(pooled, w2_ref[...], (((1,), (0,)), ((), ())),
                        preferred_element_type=jnp.float32)
    x = jnp.maximum(x + b2_ref[...], 0.0)
    pred = lax.dot_general(x, w3_ref[...], (((1,), (0,)), ((), ())),
                           preferred_element_type=jnp.float32)
    out_ref[...] = pred + b3_ref[...]


def _make_seg_sum(n, h_dim):
    mesh = plsc.VectorSubcoreMesh(core_axis_name="c", subcore_axis_name="s")
    nw = 32
    per_w = n // nw
    nchunk = per_w // _CHUNK
    rows_per_sub = _SP // 16

    @functools.partial(
        pl.kernel, mesh=mesh,
        out_type=jax.ShapeDtypeStruct((2, _SP, h_dim), jnp.float32),
        scratch_types=[
            pltpu.VMEM((_CHUNK,), jnp.int32),
            pltpu.VMEM((_CHUNK, h_dim), jnp.float32),
            pltpu.VMEM((128, h_dim), jnp.float32),
            pltpu.VMEM_SHARED((_SP, h_dim), jnp.float32),
        ],
    )
    def seg(h_hbm, idx_hbm, zeros_hbm, out_hbm, idx_v, rows_v, z_v, pool_sh):
        c = lax.axis_index("c")
        sid = lax.axis_index("s")
        wid = c * 16 + sid

        pltpu.sync_copy(zeros_hbm, z_v)
        zbase = sid * rows_per_sub
        for k in range(rows_per_sub // 128):
            pltpu.sync_copy(z_v, pool_sh.at[pl.ds(zbase + k * 128, 128)])
        plsc.subcore_barrier()

        def chunk(t, _):
            base = wid * per_w + t * _CHUNK
            pltpu.sync_copy(idx_hbm.at[pl.ds(base, _CHUNK)], idx_v)
            pltpu.sync_copy(h_hbm.at[pl.ds(base, _CHUNK)], rows_v)
            pltpu.sync_copy(rows_v, pool_sh.at[idx_v], add=True)
            return _

        lax.fori_loop(0, nchunk, chunk, None)
        plsc.subcore_barrier()

        pltpu.sync_copy(pool_sh.at[pl.ds(zbase, rows_per_sub)],
                        out_hbm.at[c].at[pl.ds(zbase, rows_per_sub)])

    return seg


def kernel(mol_embeddings, concentrations, roles, batch_indices,
           W1, b1, W2, b2, W3, b3):
    n, d = mol_embeddings.shape
    h_dim = W1.shape[1]
    s = _NUM_SEGMENTS
    bsz = 2000
    nb = n // bsz

    w1e = W1[:d].astype(jnp.bfloat16)
    wcr = W1[d:]
    b1r = b1.reshape(1, h_dim)
    b2r = b2.reshape(1, h_dim)
    b3r = b3.reshape(1, 1)

    h = pl.pallas_call(
        _h_body,
        grid=(nb,),
        in_specs=[
            pl.BlockSpec((bsz, d), lambda i: (i, 0)),
            pl.BlockSpec((bsz, 1), lambda i: (i, 0)),
            pl.BlockSpec((bsz, 1), lambda i: (i, 0)),
            pl.BlockSpec((d, h_dim), lambda i: (0, 0)),
            pl.BlockSpec((2, h_dim), lambda i: (0, 0)),
            pl.BlockSpec((1, h_dim), lambda i: (0, 0)),
        ],
        out_specs=pl.BlockSpec((bsz, h_dim), lambda i: (i, 0)),
        out_shape=jax.ShapeDtypeStruct((n, h_dim), jnp.float32),
    )(mol_embeddings, concentrations, roles, w1e, wcr, b1r)

    zeros = jnp.zeros((128, h_dim), jnp.float32)
    parts = _make_seg_sum(n, h_dim)(h, batch_indices, zeros)

    out = pl.pallas_call(
        functools.partial(_mlp_body, s=s),
        in_specs=[
            pl.BlockSpec((2, _SP, h_dim), lambda: (0, 0, 0)),
            pl.BlockSpec((h_dim, h_dim), lambda: (0, 0)),
            pl.BlockSpec((1, h_dim), lambda: (0, 0)),
            pl.BlockSpec((h_dim, 1), lambda: (0, 0)),
            pl.BlockSpec((1, 1), lambda: (0, 0)),
        ],
        out_specs=pl.BlockSpec((s, 1), lambda: (0, 0)),
        out_shape=jax.ShapeDtypeStruct((s, 1), jnp.float32),
    )(parts, W2, b2r, W3, b3r)
    return out[:, 0]


# speedup vs baseline: 1.0664x; 1.0664x over previous
import functools

import jax
import jax.numpy as jnp
from jax import lax
from jax.experimental import pallas as pl
from jax.experimental.pallas import tpu as pltpu

_NUM_SEGMENTS = 10000
_WINDOW = 64


def _pick_block(n):
    for b in (640, 512, 320, 256, 128, 64, 32, 16, 8):
        if n % b == 0:
            return b
    return 1


def _body(emb_ref, conc_ref, roles_ref, idx_ref, w1e_ref, wcr_ref, b1_ref,
          w2_ref, b2_ref, w3_ref, b3_ref, out_ref, acc_ref, accb_ref,
          *, nb, bsz, s):
    i = pl.program_id(0)
    w = _WINDOW
    hb = bsz // 2

    @pl.when(i == 0)
    def _init():
        acc_ref[...] = jnp.zeros_like(acc_ref)
        accb_ref[...] = jnp.zeros_like(accb_ref)

    h = lax.dot_general(emb_ref[...].astype(jnp.bfloat16), w1e_ref[...],
                        (((1,), (0,)), ((), ())),
                        preferred_element_type=jnp.float32)
    h = h + conc_ref[...] * wcr_ref[0:1, :] + roles_ref[...] * wcr_ref[1:2, :]
    h = jnp.maximum(h + b1_ref[...], 0.0).astype(jnp.bfloat16)

    idx = idx_ref[...].reshape(1, bsz)
    idx_a = idx[:, :hb]
    idx_b = idx[:, hb:]
    h_a = h[:hb]
    h_b = h[hb:]
    pos = lax.broadcasted_iota(jnp.int32, (1, hb), 1)
    row = lax.broadcasted_iota(jnp.int32, (w, hb), 0)

    base_a0 = (jnp.min(idx_a) // 8) * 8
    base_b0 = (jnp.min(idx_b) // 8) * 8
    span_a = jnp.max(idx_a) - base_a0
    span_b = jnp.max(idx_b) - base_b0
    fast = (span_a < w) & (span_b < w)

    @pl.when(fast)
    def _fast():
        oh_a = jnp.where(jnp.broadcast_to(idx_a - base_a0, (w, hb)) == row,
                         1.0, 0.0).astype(jnp.bfloat16)
        oh_b = jnp.where(jnp.broadcast_to(idx_b - base_b0, (w, hb)) == row,
                         1.0, 0.0).astype(jnp.bfloat16)
        part_a = lax.dot_general(oh_a, h_a, (((1,), (0,)), ((), ())),
                                 preferred_element_type=jnp.float32)
        part_b = lax.dot_general(oh_b, h_b, (((1,), (0,)), ((), ())),
                                 preferred_element_type=jnp.float32)
        acc_ref[pl.ds(base_a0, w), :] = acc_ref[pl.ds(base_a0, w), :] + part_a
        accb_ref[pl.ds(base_b0, w), :] = accb_ref[pl.ds(base_b0, w), :] + part_b

    @pl.when(jnp.logical_not(fast))
    def _slow():
        def half_step(p, idx_h, h_h):
            live1 = pos >= p
            base = jnp.min(jnp.where(live1, idx_h, s))
            base = (base // 8) * 8
            ml = jnp.where(live1, idx_h - base, -1)
            oh = jnp.where(jnp.broadcast_to(ml, (w, hb)) == row,
                           1.0, 0.0).astype(jnp.bfloat16)
            part = lax.dot_general(oh, h_h, (((1,), (0,)), ((), ())),
                                   preferred_element_type=jnp.float32)
            cnt = jnp.sum(((ml >= 0) & (ml < w)).astype(jnp.int32))
            return base, part, cnt

        def cond(c):
            return (c[0] < hb) | (c[1] < hb)

        def step(c):
            pa, pb = c
            base_a, part_a, cnt_a = half_step(pa, idx_a, h_a)
            base_b, part_b, cnt_b = half_step(pb, idx_b, h_b)
            acc_ref[pl.ds(base_a, w), :] = (
                acc_ref[pl.ds(base_a, w), :] + part_a)
            accb_ref[pl.ds(base_b, w), :] = (
                accb_ref[pl.ds(base_b, w), :] + part_b)
            return (pa + cnt_a, pb + cnt_b)

        lax.while_loop(cond, step, (jnp.int32(0), jnp.int32(0)))

    @pl.when(i == nb - 1)
    def _tail():
        pooled = acc_ref[0:s, :] + accb_ref[0:s, :]
        x = lax.dot_general(pooled, w2_ref[...], (((1,), (0,)), ((), ())),
                            preferred_element_type=jnp.float32)
        x = jnp.maximum(x + b2_ref[...], 0.0)
        pred = lax.dot_general(x, w3_ref[...], (((1,), (0,)), ((), ())),
                               preferred_element_type=jnp.float32)
        out_ref[...] = pred + b3_ref[...]


def kernel(mol_embeddings, concentrations, roles, batch_indices,
           W1, b1, W2, b2, W3, b3):
    n, d = mol_embeddings.shape
    h_dim = W1.shape[1]
    s = _NUM_SEGMENTS
    bsz = _pick_block(n)
    nb = n // bsz
    sp = ((s + _WINDOW + 7) // 8) * 8

    idx3 = batch_indices.reshape(nb, 1, bsz)
    w1e = W1[:d].astype(jnp.bfloat16)
    wcr = W1[d:]
    b1r = b1.reshape(1, h_dim)
    b2r = b2.reshape(1, h_dim)
    b3r = b3.reshape(1, 1)

    out = pl.pallas_call(
        functools.partial(_body, nb=nb, bsz=bsz, s=s),
        grid=(nb,),
        in_specs=[
            pl.BlockSpec((bsz, d), lambda i: (i, 0)),
            pl.BlockSpec((bsz, 1), lambda i: (i, 0)),
            pl.BlockSpec((bsz, 1), lambda i: (i, 0)),
            pl.BlockSpec((1, 1, bsz), lambda i: (i, 0, 0)),
            pl.BlockSpec((d, h_dim), lambda i: (0, 0)),
            pl.BlockSpec((2, h_dim), lambda i: (0, 0)),
            pl.BlockSpec((1, h_dim), lambda i: (0, 0)),
            pl.BlockSpec((h_dim, h_dim), lambda i: (0, 0)),
            pl.BlockSpec((1, h_dim), lambda i: (0, 0)),
            pl.BlockSpec((h_dim, 1), lambda i: (0, 0)),
            pl.BlockSpec((1, 1), lambda i: (0, 0)),
        ],
        out_specs=pl.BlockSpec((s, 1), lambda i: (0, 0)),
        out_shape=jax.ShapeDtypeStruct((s, 1), jnp.float32),
        scratch_shapes=[pltpu.VMEM((sp + _WINDOW, h_dim), jnp.float32),
                        pltpu.VMEM((sp + _WINDOW, h_dim), jnp.float32)],
    )(mol_embeddings, concentrations, roles, idx3, w1e, wcr, b1r,
      W2, b2r, W3, b3r)
    return out[:, 0]
